# Initial kernel scaffold; baseline (speedup 1.0000x reference)
import jax
import jax.numpy as jnp
from jax.experimental import pallas as pl
from jax.experimental.pallas import tpu as pltpu

_PREFIX0 = 3681
_PREFIX1 = 25
_LABEL0 = 3967
_LABEL1 = 4633
_WINDOW = 10
_NEG = jnp.float32(-10000.0)


def _mask_body(ids_ref, attn_ref, out_ref):
    _, _, bq, s = attn_ref.shape
    qb = pl.program_id(0)

    ids = ids_ref[...]
    c0 = (ids == _PREFIX0).astype(jnp.int32)
    c1 = (ids == _PREFIX1).astype(jnp.int32)
    c2 = ((ids == _LABEL0) | (ids == _LABEL1)).astype(jnp.int32)
    c0s = pltpu.roll(c0, 2, axis=1)
    c1s = pltpu.roll(c1, 1, axis=1)
    p_lane = jax.lax.broadcasted_iota(jnp.int32, (1, s), 1)
    m = (c0s + c1s + c2 + (p_lane >= 2).astype(jnp.int32)) == 4

    q_idx = jax.lax.broadcasted_iota(jnp.int32, (bq, s), 0) + qb * bq
    p_idx = jax.lax.broadcasted_iota(jnp.int32, (bq, s), 1)
    win = jnp.broadcast_to(m, (bq, s)) & (p_idx <= q_idx) & (p_idx >= q_idx - _WINDOW)
    t = jnp.max(jnp.where(win, p_idx, -1), axis=1, keepdims=True)
    mask = p_idx < t

    blk = attn_ref[...]
    out_ref[...] = jnp.where(mask[None, None, :, :], _NEG, blk)


def kernel(attn_weights, input_ids):
    b, h, s, _ = attn_weights.shape
    ids32 = input_ids.astype(jnp.int32)
    bq = 128
    grid = (s // bq,)
    out = pl.pallas_call(
        _mask_body,
        grid=grid,
        in_specs=[
            pl.BlockSpec((1, s), lambda i: (0, 0)),
            pl.BlockSpec((1, h, bq, s), lambda i: (0, 0, i, 0)),
        ],
        out_specs=pl.BlockSpec((1, h, bq, s), lambda i: (0, 0, i, 0)),
        out_shape=jax.ShapeDtypeStruct((b, h, s, s), jnp.float32),
        compiler_params=pltpu.CompilerParams(
            dimension_semantics=("arbitrary",),
        ),
    )(ids32, attn_weights)
    return out


# speedup vs baseline: 324.9310x; 324.9310x over previous
import jax
import jax.numpy as jnp
from jax.experimental import pallas as pl
from jax.experimental.pallas import tpu as pltpu

_PREFIX0 = 3681
_PREFIX1 = 25
_LABEL0 = 3967
_LABEL1 = 4633
_WINDOW = 10
_NEG = -10000.0


def _mask_body(ids_ref, attn_ref, out_ref):
    _, _, bq, s = attn_ref.shape
    qb = pl.program_id(0)

    ids = ids_ref[...]
    c0 = (ids == _PREFIX0).astype(jnp.int32)
    c1 = (ids == _PREFIX1).astype(jnp.int32)
    c2 = ((ids == _LABEL0) | (ids == _LABEL1)).astype(jnp.int32)
    c0s = pltpu.roll(c0, jnp.int32(2), axis=1)
    c1s = pltpu.roll(c1, jnp.int32(1), axis=1)
    p_lane = jax.lax.broadcasted_iota(jnp.int32, (1, s), 1)
    m = (c0s + c1s + c2 + (p_lane >= 2).astype(jnp.int32)) == 4

    q_idx = jax.lax.broadcasted_iota(jnp.int32, (bq, s), 0) + qb * bq
    p_idx = jax.lax.broadcasted_iota(jnp.int32, (bq, s), 1)
    win = jnp.broadcast_to(m, (bq, s)) & (p_idx <= q_idx) & (p_idx >= q_idx - _WINDOW)
    t = jnp.max(jnp.where(win, p_idx, -1), axis=1, keepdims=True)
    mask = p_idx < t

    blk = attn_ref[...]
    out_ref[...] = jnp.where(mask[None, None, :, :], _NEG, blk)


def kernel(attn_weights, input_ids):
    b, h, s, _ = attn_weights.shape
    ids32 = input_ids.astype(jnp.int32)
    bq = 128
    grid = (s // bq,)
    with jax.enable_x64(False):
        out = _call(attn_weights, ids32, b, h, s, bq, grid)
    return out


def _call(attn_weights, ids32, b, h, s, bq, grid):
    out = pl.pallas_call(
        _mask_body,
        grid=grid,
        in_specs=[
            pl.BlockSpec((1, s), lambda i: (0, 0)),
            pl.BlockSpec((1, h, bq, s), lambda i: (0, 0, i, 0)),
        ],
        out_specs=pl.BlockSpec((1, h, bq, s), lambda i: (0, 0, i, 0)),
        out_shape=jax.ShapeDtypeStruct((b, h, s, s), jnp.float32),
        compiler_params=pltpu.CompilerParams(
            dimension_semantics=("arbitrary",),
        ),
    )(ids32, attn_weights)
    return out
